# Initial kernel scaffold; baseline (speedup 1.0000x reference)
import jax
import jax.numpy as jnp
from jax.experimental import pallas as pl


def kernel(x, attr_emb, option_tables, prior):
    raise NotImplementedError("write your pallas kernel here")


# speedup vs baseline: 9.2819x; 9.2819x over previous
import functools

import jax
import jax.numpy as jnp
from jax import lax
from jax.experimental import pallas as pl
from jax.experimental.pallas import tpu as pltpu
from jax.experimental.pallas import tpu_sc as plsc


def _sc_gather(table2d, idx):
    num_rows = idx.shape[0]
    wide = table2d.shape[1]
    window = 512
    info = plsc.get_sparse_core_info()
    nc, ns = info.num_cores, info.num_subcores
    nw = nc * ns
    n_chunks = num_rows // (nw * window)
    assert num_rows == n_chunks * nw * window
    mesh = plsc.VectorSubcoreMesh(core_axis_name="c", subcore_axis_name="s")

    @functools.partial(
        pl.kernel,
        out_type=jax.ShapeDtypeStruct((num_rows, wide), jnp.float32),
        mesh=mesh,
        scratch_types=[
            pltpu.VMEM((window,), jnp.int32),
            pltpu.VMEM((window, wide), jnp.float32),
            pltpu.SemaphoreType.DMA,
        ],
    )
    def gather_kernel(tbl_hbm, idx_hbm, out_hbm, idx_v, rows_v, sem):
        wid = lax.axis_index("s") * nc + lax.axis_index("c")

        @pl.loop(0, n_chunks)
        def _(c):
            base = (wid * n_chunks + c) * window
            pltpu.sync_copy(idx_hbm.at[pl.ds(base, window)], idx_v)
            pltpu.async_copy(tbl_hbm.at[idx_v], rows_v, sem).wait()
            pltpu.sync_copy(rows_v, out_hbm.at[pl.ds(base, window)])

    return gather_kernel(table2d, idx)


def _epilogue_kernel(g_ref, x_ref, emb_ref, prior_ref, out_ref):
    g = g_ref[...]
    left = g[:, :, :64]
    right = g[:, :, 64:]
    odd = (x_ref[...] & 1)[:, :, None] == 1
    val = jnp.where(odd, right, left)
    out_ref[...] = (val + emb_ref[...]) * prior_ref[...]


def _epilogue(gathered, x, attr_emb, prior):
    batch, n_attrs = x.shape
    d_model = attr_emb.shape[1]
    g3 = gathered.reshape(batch, n_attrs, 2 * d_model)
    bb = 32
    return pl.pallas_call(
        _epilogue_kernel,
        grid=(batch // bb,),
        in_specs=[
            pl.BlockSpec((bb, n_attrs, 2 * d_model), lambda i: (i, 0, 0)),
            pl.BlockSpec((bb, n_attrs), lambda i: (i, 0)),
            pl.BlockSpec((1, n_attrs, d_model), lambda i: (0, 0, 0)),
            pl.BlockSpec((1, n_attrs, 1), lambda i: (0, 0, 0)),
        ],
        out_specs=pl.BlockSpec((bb, n_attrs, d_model), lambda i: (i, 0, 0)),
        out_shape=jax.ShapeDtypeStruct((batch, n_attrs, d_model), jnp.float32),
    )(g3, x, attr_emb.reshape(1, n_attrs, d_model), prior.reshape(1, n_attrs, 1))


def kernel(x, attr_emb, option_tables, prior):
    batch, n_attrs = x.shape
    _, n_options, d_model = option_tables.shape

    pairs = option_tables.reshape(n_attrs * n_options // 2, 2 * d_model)
    offs = (jnp.arange(n_attrs, dtype=jnp.int32) * n_options)[None, :]
    pair_idx = ((x + offs) >> 1).reshape(batch * n_attrs)

    gathered = _sc_gather(pairs, pair_idx)
    return _epilogue(gathered, x, attr_emb, prior)


# speedup vs baseline: 19.0762x; 2.0552x over previous
import functools

import jax
import jax.numpy as jnp
from jax import lax
from jax.experimental import pallas as pl
from jax.experimental.pallas import tpu as pltpu
from jax.experimental.pallas import tpu_sc as plsc


def _fold_kernel(tbl_ref, emb_ref, prior_ref, out_ref):
    v = (tbl_ref[...] + emb_ref[...]) * prior_ref[...]
    out_ref[...] = jnp.concatenate([v, v], axis=-1).reshape(out_ref.shape)


def _fold_tables(option_tables, attr_emb, prior):
    n_attrs, n_options, d_model = option_tables.shape
    ab = 4
    return pl.pallas_call(
        _fold_kernel,
        grid=(n_attrs // ab,),
        in_specs=[
            pl.BlockSpec((ab, n_options, d_model), lambda i: (i, 0, 0)),
            pl.BlockSpec((ab, 1, d_model), lambda i: (i, 0, 0)),
            pl.BlockSpec((ab, 1, 1), lambda i: (i, 0, 0)),
        ],
        out_specs=pl.BlockSpec((ab * n_options, 2 * d_model), lambda i: (i, 0)),
        out_shape=jax.ShapeDtypeStruct((n_attrs * n_options, 2 * d_model), jnp.float32),
    )(option_tables, attr_emb.reshape(n_attrs, 1, d_model), prior.reshape(n_attrs, 1, 1))


def _sc_gather(table2d, idx, window=400):
    num_rows = idx.shape[0]
    wide = table2d.shape[1]
    info = plsc.get_sparse_core_info()
    nc, ns = info.num_cores, info.num_subcores
    nw = nc * ns
    n_chunks = num_rows // (nw * window)
    assert num_rows == n_chunks * nw * window
    mesh = plsc.VectorSubcoreMesh(core_axis_name="c", subcore_axis_name="s")

    @functools.partial(
        pl.kernel,
        out_type=jax.ShapeDtypeStruct((num_rows, wide), jnp.float32),
        mesh=mesh,
        scratch_types=[
            pltpu.VMEM((window,), jnp.int32),
            pltpu.VMEM((window, wide), jnp.float32),
            pltpu.SemaphoreType.DMA,
        ],
    )
    def gather_kernel(tbl_hbm, idx_hbm, out_hbm, idx_v, rows_v, sem):
        wid = lax.axis_index("s") * nc + lax.axis_index("c")

        @pl.loop(0, n_chunks)
        def _(c):
            base = (c * nw + wid) * window
            pltpu.sync_copy(idx_hbm.at[pl.ds(base, window)], idx_v)
            pltpu.async_copy(tbl_hbm.at[idx_v], rows_v, sem).wait()
            pltpu.sync_copy(rows_v, out_hbm.at[pl.ds(base, window)])

    return gather_kernel(table2d, idx)


def _epilogue_kernel(g_ref, out_ref):
    val = g_ref[...][:, :64]
    eye = jnp.eye(64, dtype=jnp.float32)
    out_ref[0] = jax.lax.dot_general(
        eye, val, (((1,), (1,)), ((), ())),
        preferred_element_type=jnp.float32)


def _epilogue_half_kernel(g_ref, alias_ref, out_ref):
    del alias_ref
    _epilogue_kernel(g_ref, out_ref)


def _epilogue_half(gathered_half, prev, a0, n_half, batch, n_attrs, d_model):
    out_shape = jax.ShapeDtypeStruct((n_attrs, d_model, batch), jnp.float32)
    out_spec = pl.BlockSpec((1, d_model, batch), lambda a: (a0 + a, 0, 0))
    g_spec = pl.BlockSpec((batch, 2 * d_model), lambda a: (a, 0))
    if prev is None:
        return pl.pallas_call(
            _epilogue_kernel,
            grid=(n_half,),
            in_specs=[g_spec],
            out_specs=out_spec,
            out_shape=out_shape,
        )(gathered_half)
    return pl.pallas_call(
        _epilogue_half_kernel,
        grid=(n_half,),
        in_specs=[g_spec, pl.BlockSpec((1, 8, 128), lambda a: (0, 0, 0))],
        out_specs=out_spec,
        out_shape=out_shape,
        input_output_aliases={1: 0},
    )(gathered_half, prev)


def kernel(x, attr_emb, option_tables, prior):
    batch, n_attrs = x.shape
    _, n_options, d_model = option_tables.shape
    half = n_attrs // 2

    table = _fold_tables(option_tables, attr_emb, prior)
    xt = x.T
    offs = (jnp.arange(n_attrs, dtype=jnp.int32) * n_options)[:, None]
    flat_idx = (xt + offs).reshape(n_attrs, batch)

    g1 = _sc_gather(table, flat_idx[:half].reshape(half * batch))
    g2 = _sc_gather(table, flat_idx[half:].reshape(half * batch))
    out1 = _epilogue_half(g1, None, 0, half, batch, n_attrs, d_model)
    out2 = _epilogue_half(g2, out1, half, half, batch, n_attrs, d_model)
    return jnp.transpose(out2, (2, 0, 1))
